# baseline (speedup 1.0000x reference)
import functools

import jax
import jax.numpy as jnp
from jax import lax
from jax.experimental import pallas as pl
from jax.experimental.pallas import tpu as pltpu
from jax.experimental.pallas import tpu_sc as plsc

NSEG = 256
D = 128
N_ROWS = 100000
NC = 2
NS = 16
NW = NC * NS
CH = 128
N_FULL = 24
T_BIG = 56
T_SMALL = 48
BIG_WORKERS = 20
NBUF = 3

_mesh = plsc.VectorSubcoreMesh(core_axis_name="c", subcore_axis_name="s")


@functools.partial(
    pl.kernel,
    mesh=_mesh,
    compiler_params=pltpu.CompilerParams(needs_layout_passes=False),
    out_type=jax.ShapeDtypeStruct((NC, NSEG, D), jnp.float32),
    scratch_types=(
        [pltpu.VMEM((CH,), jnp.int32) for _ in range(N_FULL)]
        + [pltpu.VMEM((CH, D), jnp.float32) for _ in range(NBUF)]
        + [
            pltpu.VMEM((NSEG, D), jnp.float32),
            pltpu.VMEM((T_BIG,), jnp.int32),
            pltpu.VMEM((T_SMALL,), jnp.int32),
            pltpu.VMEM((T_BIG, D), jnp.float32),
            pltpu.VMEM((T_SMALL, D), jnp.float32),
            pltpu.VMEM((NSEG // 2,), jnp.int32),
            pltpu.VMEM((NSEG // 2,), jnp.int32),
            pltpu.VMEM((NS, D), jnp.float32),
            pltpu.VMEM_SHARED((NSEG, D), jnp.float32),
            pltpu.SemaphoreType.DMA,
            pltpu.SemaphoreType.DMA,
        ]
        + [pltpu.SemaphoreType.DMA for _ in range(NBUF)]
    ),
)
def _sc_partials(feat_hbm, ids_hbm, out_hbm, *scratch):
    ids_bufs = scratch[0:N_FULL]
    row_bufs = scratch[N_FULL:N_FULL + NBUF]
    (acc_l, ids_tb, ids_ts, rows_tb, rows_ts, idn_lo, idn_hi, zbuf, acc,
     sem_ids, sem_m) = scratch[N_FULL + NBUF:N_FULL + NBUF + 11]
    sem_r = scratch[N_FULL + NBUF + 11:]

    c = lax.axis_index("c")
    s = lax.axis_index("s")
    wid = s * NC + c
    base = pl.multiple_of(3120 * wid + 8 * jnp.minimum(wid, BIG_WORKERS), 8)

    ids_copies = []
    for j in range(N_FULL):
        off = pl.multiple_of(base + j * CH, 8)
        ci = pltpu.make_async_copy(ids_hbm.at[pl.ds(off, CH)], ids_bufs[j], sem_ids)
        ci.start()
        ids_copies.append(ci)

    loads = {}

    def start_load(j):
        p = j % NBUF
        off = pl.multiple_of(base + j * CH, 8)
        cr = pltpu.make_async_copy(feat_hbm.at[pl.ds(off, CH), :], row_bufs[p], sem_r[p])
        cr.start()
        loads[j] = cr

    for j in range(NBUF):
        start_load(j)

    zero = jnp.zeros((16,), jnp.float32)
    iota = lax.iota(jnp.int32, 16)
    for i in range(NS):
        for j in range(D // 16):
            zbuf[i, pl.ds(j * 16, 16)] = zero
    pltpu.sync_copy(zbuf, acc.at[pl.ds(s * NS, NS), :])
    plsc.subcore_barrier()
    for k in range(NSEG // 32):
        idn_lo[pl.ds(k * 16, 16)] = iota + (k * 16)
        idn_hi[pl.ds(k * 16, 16)] = iota + (NSEG // 2 + k * 16)

    def zero_acc_rows(i, carry):
        for r in range(8):
            for jc in range(D // 16):
                acc_l[i * 8 + r, pl.ds(jc * 16, 16)] = zero
        return carry

    lax.fori_loop(0, NSEG // 8, zero_acc_rows, 0)

    for ci in ids_copies:
        ci.wait()

    col_idx = [iota + (jc * 16) for jc in range(D // 16)]

    for j in range(N_FULL):
        p = j % NBUF
        loads.pop(j).wait()
        rows_v = row_bufs[p]
        ids_v = ids_bufs[j]

        def accum_block(i, carry, rows_v=rows_v, ids_v=ids_v):
            ids16 = ids_v[pl.ds(i * 16, 16)]
            for r in range(16):
                rid = ids16.at[jnp.full((16,), r, jnp.int32)].get(
                    mode="promise_in_bounds")
                for jc in range(D // 16):
                    vals = rows_v[i * 16 + r, pl.ds(jc * 16, 16)]
                    plsc.addupdate_scatter(acc_l, [rid, col_idx[jc]], vals)
            return carry

        lax.fori_loop(0, CH // 16, accum_block, 0)
        nxt = j + NBUF
        if nxt < N_FULL:
            start_load(nxt)

    toff = pl.multiple_of(base + N_FULL * CH, 8)

    @pl.when(wid < BIG_WORKERS)
    def _big_tail():
        pltpu.sync_copy(ids_hbm.at[pl.ds(toff, T_BIG)], ids_tb)
        pltpu.sync_copy(feat_hbm.at[pl.ds(toff, T_BIG), :], rows_tb)
        pltpu.sync_copy(rows_tb, acc.at[ids_tb], add=True)

    @pl.when(wid >= BIG_WORKERS)
    def _small_tail():
        pltpu.sync_copy(ids_hbm.at[pl.ds(toff, T_SMALL)], ids_ts)
        pltpu.sync_copy(feat_hbm.at[pl.ds(toff, T_SMALL), :], rows_ts)
        pltpu.sync_copy(rows_ts, acc.at[ids_ts], add=True)

    m_lo = pltpu.make_async_copy(acc_l.at[pl.ds(0, NSEG // 2), :],
                                 acc.at[idn_lo], sem_m)
    m_lo.start(add=True)
    m_hi = pltpu.make_async_copy(acc_l.at[pl.ds(NSEG // 2, NSEG // 2), :],
                                 acc.at[idn_hi], sem_m)
    m_hi.start(add=True)
    m_lo.wait()
    m_hi.wait()

    plsc.subcore_barrier()
    pltpu.sync_copy(
        acc.at[pl.ds(s * NS, NS), :],
        out_hbm.at[c, pl.ds(s * NS, NS), :],
    )


def _combine(partials):
    def body(p_ref, o_ref):
        o_ref[...] = p_ref[0, :, :] + p_ref[1, :, :]

    return pl.pallas_call(
        body,
        out_shape=jax.ShapeDtypeStruct((NSEG, D), jnp.float32),
    )(partials)


def kernel(feat, segment_ids):
    partials = _sc_partials(feat, segment_ids.astype(jnp.int32))
    return _combine(partials)


# speedup vs baseline: 1.1390x; 1.1390x over previous
import functools

import jax
import jax.numpy as jnp
from jax import lax
from jax.experimental import pallas as pl
from jax.experimental.pallas import tpu as pltpu
from jax.experimental.pallas import tpu_sc as plsc

NSEG = 256
D = 128
N_ROWS = 100000
NC = 2
NS = 16
NW = NC * NS

S_TC = 36000
BLK = 288
NBLK = S_TC // BLK

ROWS_SC = N_ROWS - S_TC
PER_W = ROWS_SC // NW
CH = 128
N_FULL = PER_W // CH
TAIL = PER_W - N_FULL * CH
NBUF = 3

_mesh = plsc.VectorSubcoreMesh(core_axis_name="c", subcore_axis_name="s")


@functools.partial(
    pl.kernel,
    mesh=_mesh,
    out_type=jax.ShapeDtypeStruct((NC, NSEG, D), jnp.float32),
    scratch_types=(
        [pltpu.VMEM((CH,), jnp.int32) for _ in range(NBUF)]
        + [pltpu.VMEM((CH, D), jnp.float32) for _ in range(NBUF)]
        + [
            pltpu.VMEM((TAIL,), jnp.int32),
            pltpu.VMEM((TAIL, D), jnp.float32),
            pltpu.VMEM((NS, D), jnp.float32),
            pltpu.VMEM_SHARED((NSEG, D), jnp.float32),
        ]
        + [pltpu.SemaphoreType.DMA for _ in range(3 * NBUF)]
    ),
)
def _sc_partials(feat_hbm, ids_hbm, out_hbm, *scratch):
    ids_bufs = scratch[0:NBUF]
    row_bufs = scratch[NBUF:2 * NBUF]
    ids_t, rows_t, zbuf, acc = scratch[2 * NBUF:2 * NBUF + 4]
    sems = scratch[2 * NBUF + 4:]
    sem_i = sems[0:NBUF]
    sem_r = sems[NBUF:2 * NBUF]
    sem_s = sems[2 * NBUF:3 * NBUF]

    c = lax.axis_index("c")
    s = lax.axis_index("s")
    wid = s * NC + c
    base = pl.multiple_of(S_TC + PER_W * wid, 8)

    loads = {}

    def start_load(j):
        p = j % NBUF
        off = pl.multiple_of(base + j * CH, 8)
        ci = pltpu.make_async_copy(ids_hbm.at[pl.ds(off, CH)], ids_bufs[p], sem_i[p])
        cr = pltpu.make_async_copy(feat_hbm.at[pl.ds(off, CH), :], row_bufs[p], sem_r[p])
        ci.start()
        cr.start()
        loads[j] = (ci, cr)

    start_load(0)

    zero = jnp.zeros((16,), jnp.float32)
    for i in range(NS):
        for j in range(D // 16):
            zbuf[i, pl.ds(j * 16, 16)] = zero
    pltpu.sync_copy(zbuf, acc.at[pl.ds(s * NS, NS), :])
    plsc.subcore_barrier()

    for j in range(1, NBUF - 1):
        start_load(j)

    scats = {}
    for j in range(N_FULL):
        p = j % NBUF
        ci, cr = loads.pop(j)
        ci.wait()
        cr.wait()
        if j >= 1:
            scats.pop(j - 1).wait()
        sc = pltpu.make_async_copy(row_bufs[p], acc.at[ids_bufs[p]], sem_s[p])
        sc.start(add=True)
        scats[j] = sc
        nxt = j + NBUF - 1
        if nxt < N_FULL:
            start_load(nxt)
    scats.pop(N_FULL - 1).wait()

    toff = pl.multiple_of(base + N_FULL * CH, 8)
    pltpu.sync_copy(ids_hbm.at[pl.ds(toff, TAIL)], ids_t)
    pltpu.sync_copy(feat_hbm.at[pl.ds(toff, TAIL), :], rows_t)
    pltpu.sync_copy(rows_t, acc.at[ids_t], add=True)

    plsc.subcore_barrier()
    pltpu.sync_copy(
        acc.at[pl.ds(s * NS, NS), :],
        out_hbm.at[c, pl.ds(s * NS, NS), :],
    )


def _tc_matmul(feat_tc, ids_tc3):
    def body(ids_ref, feat_ref, o_ref):
        @pl.when(pl.program_id(0) == 0)
        def _init():
            o_ref[...] = jnp.zeros_like(o_ref)

        seg = lax.broadcasted_iota(jnp.int32, (NSEG, BLK), 0)
        onehot = (seg == ids_ref[0]).astype(jnp.float32)
        o_ref[...] += lax.dot_general(
            onehot, feat_ref[...], (((1,), (0,)), ((), ())),
            preferred_element_type=jnp.float32)

    return pl.pallas_call(
        body,
        grid=(NBLK,),
        in_specs=[
            pl.BlockSpec((1, 1, BLK), lambda i: (i, 0, 0)),
            pl.BlockSpec((BLK, D), lambda i: (i, 0)),
        ],
        out_specs=pl.BlockSpec((NSEG, D), lambda i: (0, 0)),
        out_shape=jax.ShapeDtypeStruct((NSEG, D), jnp.float32),
    )(ids_tc3, feat_tc)


def _combine(partials, tc_out):
    def body(p_ref, t_ref, o_ref):
        o_ref[...] = p_ref[0, :, :] + p_ref[1, :, :] + t_ref[...]

    return pl.pallas_call(
        body,
        out_shape=jax.ShapeDtypeStruct((NSEG, D), jnp.float32),
    )(partials, tc_out)


def kernel(feat, segment_ids):
    ids32 = segment_ids.astype(jnp.int32)
    partials = _sc_partials(feat, ids32)
    ids_tc3 = ids32[:S_TC].reshape(NBLK, 1, BLK)
    tc_out = _tc_matmul(feat[:S_TC], ids_tc3)
    return _combine(partials, tc_out)


# speedup vs baseline: 1.7253x; 1.5147x over previous
import functools

import jax
import jax.numpy as jnp
from jax import lax
from jax.experimental import pallas as pl
from jax.experimental.pallas import tpu as pltpu
from jax.experimental.pallas import tpu_sc as plsc

NSEG = 256
D = 128
N_ROWS = 100000
NC = 2
NS = 16
NW = NC * NS
CH = 256
IDC = 128
N_FULL = 12
T_BIG = 56
T_SMALL = 48
BIG_WORKERS = 20
NBUF = 3

_mesh = plsc.VectorSubcoreMesh(core_axis_name="c", subcore_axis_name="s")


@functools.partial(
    pl.kernel,
    mesh=_mesh,
    out_type=jax.ShapeDtypeStruct((NC, NSEG, D), jnp.float32),
    scratch_types=(
        [pltpu.VMEM((IDC,), jnp.int32) for _ in range(2 * NBUF)]
        + [pltpu.VMEM((CH, D), jnp.float32) for _ in range(NBUF)]
        + [
            pltpu.VMEM((T_BIG,), jnp.int32),
            pltpu.VMEM((T_SMALL,), jnp.int32),
            pltpu.VMEM((T_BIG, D), jnp.float32),
            pltpu.VMEM((T_SMALL, D), jnp.float32),
            pltpu.VMEM((NS, D), jnp.float32),
            pltpu.VMEM_SHARED((NSEG, D), jnp.float32),
        ]
        + [pltpu.SemaphoreType.DMA for _ in range(3 * NBUF)]
    ),
)
def _sc_partials(feat_hbm, ids_hbm, out_hbm, *scratch):
    ids_bufs = scratch[0:2 * NBUF]
    row_bufs = scratch[2 * NBUF:3 * NBUF]
    ids_tb, ids_ts, rows_tb, rows_ts, zbuf, acc = scratch[3 * NBUF:3 * NBUF + 6]
    sems = scratch[3 * NBUF + 6:]
    sem_i = sems[0:NBUF]
    sem_r = sems[NBUF:2 * NBUF]
    sem_s = sems[2 * NBUF:3 * NBUF]

    c = lax.axis_index("c")
    s = lax.axis_index("s")
    wid = s * NC + c
    base = pl.multiple_of(3120 * wid + 8 * jnp.minimum(wid, BIG_WORKERS), 8)

    loads = {}

    def start_load(j):
        p = j % NBUF
        off = pl.multiple_of(base + j * CH, 8)
        ci0 = pltpu.make_async_copy(
            ids_hbm.at[pl.ds(off, IDC)], ids_bufs[2 * p], sem_i[p])
        ci1 = pltpu.make_async_copy(
            ids_hbm.at[pl.ds(off + IDC, IDC)], ids_bufs[2 * p + 1], sem_i[p])
        cr = pltpu.make_async_copy(feat_hbm.at[pl.ds(off, CH), :], row_bufs[p], sem_r[p])
        ci0.start()
        ci1.start()
        cr.start()
        loads[j] = (ci0, ci1, cr)

    start_load(0)

    zero = jnp.zeros((16,), jnp.float32)
    for i in range(NS):
        for j in range(D // 16):
            zbuf[i, pl.ds(j * 16, 16)] = zero
    pltpu.sync_copy(zbuf, acc.at[pl.ds(s * NS, NS), :])
    plsc.subcore_barrier()

    for j in range(1, NBUF - 1):
        start_load(j)

    scats = {}
    for j in range(N_FULL):
        p = j % NBUF
        for cp in loads.pop(j):
            cp.wait()
        if j >= 1:
            for sc in scats.pop(j - 1):
                sc.wait()
        sc0 = pltpu.make_async_copy(
            row_bufs[p].at[pl.ds(0, IDC), :], acc.at[ids_bufs[2 * p]], sem_s[p])
        sc1 = pltpu.make_async_copy(
            row_bufs[p].at[pl.ds(IDC, IDC), :], acc.at[ids_bufs[2 * p + 1]], sem_s[p])
        sc0.start(add=True)
        sc1.start(add=True)
        scats[j] = (sc0, sc1)
        nxt = j + NBUF - 1
        if nxt < N_FULL:
            start_load(nxt)
    for sc in scats.pop(N_FULL - 1):
        sc.wait()

    toff = pl.multiple_of(base + N_FULL * CH, 8)

    @pl.when(wid < BIG_WORKERS)
    def _big_tail():
        pltpu.sync_copy(ids_hbm.at[pl.ds(toff, T_BIG)], ids_tb)
        pltpu.sync_copy(feat_hbm.at[pl.ds(toff, T_BIG), :], rows_tb)
        pltpu.sync_copy(rows_tb, acc.at[ids_tb], add=True)

    @pl.when(wid >= BIG_WORKERS)
    def _small_tail():
        pltpu.sync_copy(ids_hbm.at[pl.ds(toff, T_SMALL)], ids_ts)
        pltpu.sync_copy(feat_hbm.at[pl.ds(toff, T_SMALL), :], rows_ts)
        pltpu.sync_copy(rows_ts, acc.at[ids_ts], add=True)

    plsc.subcore_barrier()
    pltpu.sync_copy(
        acc.at[pl.ds(s * NS, NS), :],
        out_hbm.at[c, pl.ds(s * NS, NS), :],
    )


def _combine(partials):
    def body(p_ref, o_ref):
        o_ref[...] = p_ref[0, :, :] + p_ref[1, :, :]

    return pl.pallas_call(
        body,
        out_shape=jax.ShapeDtypeStruct((NSEG, D), jnp.float32),
    )(partials)


def kernel(feat, segment_ids):
    partials = _sc_partials(feat, segment_ids.astype(jnp.int32))
    return _combine(partials)
